# Initial kernel scaffold; baseline (speedup 1.0000x reference)
import jax
import jax.numpy as jnp
from jax.experimental import pallas as pl


def kernel(x_connection, x_entity, edge_to, edge_rev, edge_link, W_in, b_in, Wk, bk, Wq, bq, Wv, bv, a_rel, m_rel, mu, Wa, ba, skip, W_out, b_out):
    raise NotImplementedError("write your pallas kernel here")


# speedup vs baseline: 15.2963x; 15.2963x over previous
import functools
import math

import jax
import jax.numpy as jnp
from jax import lax
from jax.experimental import pallas as pl
from jax.experimental.pallas import tpu as pltpu

_ET_TI = [(1, 0), (0, 1), (0, 0)]
_N = 50000
_HID = 128
_H = 8
_D = 16
_L = 2
_BR = 400
_GRID = _N // _BR


def _gelu(x):
    c = math.sqrt(2.0 / math.pi)
    return 0.5 * x * (1.0 + jnp.tanh(c * (x + 0.044715 * (x * x * x))))


def _mm_body(x_ref, w_ref, b_ref, o_ref, *, act):
    x = x_ref[...]
    o = jnp.dot(x, w_ref[...], preferred_element_type=jnp.float32) + b_ref[...]
    if act == 'relu':
        o = jnp.maximum(o, 0.0)
    o_ref[...] = o


def _mm(x, w, b, act=None):
    n, kdim = x.shape
    ndim = w.shape[1]
    return pl.pallas_call(
        functools.partial(_mm_body, act=act),
        grid=(_GRID,),
        in_specs=[
            pl.BlockSpec((_BR, kdim), lambda i: (i, 0)),
            pl.BlockSpec((kdim, ndim), lambda i: (0, 0)),
            pl.BlockSpec((1, ndim), lambda i: (0, 0)),
        ],
        out_specs=pl.BlockSpec((_BR, ndim), lambda i: (i, 0)),
        out_shape=jax.ShapeDtypeStruct((n, ndim), jnp.float32),
    )(x, w, b.reshape(1, ndim))


def _blend_body(out_ref, x_ref, w_ref, b_ref, beta_ref, o_ref):
    g = _gelu(out_ref[...])
    proj = jnp.dot(g, w_ref[...], preferred_element_type=jnp.float32) + b_ref[...]
    beta = beta_ref[...]
    o_ref[...] = beta * proj + (1.0 - beta) * x_ref[...]


def _blend(out, x, w, b, beta):
    n = x.shape[0]
    beta_vec = jnp.full((1, _HID), beta, jnp.float32)
    return pl.pallas_call(
        _blend_body,
        grid=(_GRID,),
        in_specs=[
            pl.BlockSpec((_BR, _HID), lambda i: (i, 0)),
            pl.BlockSpec((_BR, _HID), lambda i: (i, 0)),
            pl.BlockSpec((_HID, _HID), lambda i: (0, 0)),
            pl.BlockSpec((1, _HID), lambda i: (0, 0)),
            pl.BlockSpec((1, _HID), lambda i: (0, 0)),
        ],
        out_specs=pl.BlockSpec((_BR, _HID), lambda i: (i, 0)),
        out_shape=jax.ShapeDtypeStruct((n, _HID), jnp.float32),
    )(out, x, w, b.reshape(1, _HID), beta_vec)


def _block_diag(a):
    eye = jnp.eye(_H, dtype=a.dtype)
    out = jnp.einsum('hde,hg->hdge', a, eye)
    return out.reshape(_HID, _H * _D).reshape(_HID, _HID)


def _edge_phase_xla(q_dst, k_rel, v_rel, src, dst, mu_scale, nd):
    ke = k_rel[src].reshape(-1, _H, _D)
    ve = v_rel[src].reshape(-1, _H, _D)
    qe = q_dst[dst].reshape(-1, _H, _D)
    alpha = (qe * ke).sum(-1) * mu_scale
    amax = jax.ops.segment_max(alpha, dst, num_segments=nd)
    amax = jnp.where(jnp.isfinite(amax), amax, 0.0)
    ea = jnp.exp(alpha - amax[dst])
    den = jax.ops.segment_sum(ea, dst, num_segments=nd)
    msg = ve * ea[..., None]
    acc = jax.ops.segment_sum(msg.reshape(-1, _HID), dst, num_segments=nd)
    return acc / (jnp.repeat(den, _D, axis=1) + 1e-16)


def kernel(x_connection, x_entity, edge_to, edge_rev, edge_link, W_in, b_in,
           Wk, bk, Wq, bq, Wv, bv, a_rel, m_rel, mu, Wa, ba, skip,
           W_out, b_out):
    edges = [edge_to, edge_rev, edge_link]
    xin = [x_connection, x_entity]
    x = [_mm(xin[ti], W_in[ti], b_in[ti], act='relu') for ti in range(2)]

    inv_sqrt_d = 1.0 / math.sqrt(_D)
    for l in range(_L):
        q = [_mm(x[ti], Wq[l, ti], bq[l, ti]) for ti in range(2)]
        out = [jnp.zeros((_N, _HID), jnp.float32) for _ in range(2)]
        for ri, (sti, dti) in enumerate(_ET_TI):
            bd_a = _block_diag(a_rel[l, ri])
            bd_m = _block_diag(m_rel[l, ri])
            wkv = jnp.concatenate([Wk[l, sti] @ bd_a, Wv[l, sti] @ bd_m], axis=1)
            bkv = jnp.concatenate([bk[l, sti] @ bd_a, bv[l, sti] @ bd_m], axis=0)
            kv = _mm(x[sti], wkv, bkv)
            k_rel, v_rel = kv[:, :_HID], kv[:, _HID:]
            mu_scale = mu[l, ri] * inv_sqrt_d
            out[dti] = out[dti] + _edge_phase_xla(
                q[dti], k_rel, v_rel, edges[ri][0], edges[ri][1], mu_scale, _N)
        newx = []
        for ti in range(2):
            beta = jax.nn.sigmoid(skip[l, ti])
            newx.append(_blend(out[ti], x[ti], Wa[l, ti], ba[l, ti], beta))
        x = newx
    return _mm(x[0], W_out, b_out)
